# Initial kernel scaffold; baseline (speedup 1.0000x reference)
import functools

import jax
import jax.numpy as jnp
from jax import lax
from jax.experimental import pallas as pl
from jax.experimental.pallas import tpu as pltpu
from jax.experimental.pallas import tpu_sc as plsc

NUM_USERS = 25000
NUM_ITEMS = 25000
N = NUM_USERS + NUM_ITEMS
E = 1600000
D = 16
N_LAYERS = 3

NUM_CORES = 2
NUM_SUBCORES = 16
NUM_WORKERS = NUM_CORES * NUM_SUBCORES
CHUNK = 128
CHUNKS_PER_TILE = -(-E // (NUM_WORKERS * CHUNK))
E_PAD = NUM_WORKERS * CHUNK * CHUNKS_PER_TILE
ROWS_PER_TILE = N // NUM_SUBCORES
UNROLL = 8


def _make_layer():
  mesh = plsc.VectorSubcoreMesh(core_axis_name="c", subcore_axis_name="s")

  @functools.partial(
      pl.kernel,
      mesh=mesh,
      out_type=jax.ShapeDtypeStruct((NUM_CORES, N, D), jnp.float32),
      scratch_types=[
          pltpu.VMEM((CHUNK,), jnp.int32),
          pltpu.VMEM((CHUNK,), jnp.int32),
          pltpu.VMEM((CHUNK,), jnp.float32),
          pltpu.VMEM((CHUNK, D), jnp.float32),
          pltpu.VMEM_SHARED((N, D), jnp.float32),
          pltpu.SemaphoreType.DMA,
      ],
  )
  def layer(src_hbm, dst_hbm, w_hbm, emb_hbm, zeros_hbm, out_hbm,
            src_v, dst_v, w_v, rows_v, acc, sem):
    cid = lax.axis_index("c")
    sid = lax.axis_index("s")
    wid = sid * NUM_CORES + cid

    r0 = sid * ROWS_PER_TILE
    pltpu.sync_copy(zeros_hbm.at[pl.ds(r0, ROWS_PER_TILE)],
                    acc.at[pl.ds(r0, ROWS_PER_TILE)])
    plsc.subcore_barrier()

    def chunk_body(c, carry):
      base = (wid * CHUNKS_PER_TILE + c) * CHUNK
      pltpu.sync_copy(src_hbm.at[pl.ds(base, CHUNK)], src_v)
      pltpu.sync_copy(dst_hbm.at[pl.ds(base, CHUNK)], dst_v)
      pltpu.sync_copy(w_hbm.at[pl.ds(base, CHUNK)], w_v)
      pltpu.async_copy(emb_hbm.at[src_v], rows_v, sem).wait()

      def mul_body(g, carry2):
        for j in range(UNROLL):
          k = g * UNROLL + j
          wb = plsc.load_gather(w_v, [jnp.full((16,), k, jnp.int32)])
          rows_v[k, :] = rows_v[k, :] * wb
        return carry2

      lax.fori_loop(0, CHUNK // UNROLL, mul_body, 0, unroll=False)
      pltpu.sync_copy(rows_v, acc.at[dst_v], add=True)
      return carry

    lax.fori_loop(0, CHUNKS_PER_TILE, chunk_body, 0, unroll=False)
    plsc.subcore_barrier()

    pltpu.sync_copy(acc.at[pl.ds(r0, ROWS_PER_TILE)],
                    out_hbm.at[cid, pl.ds(r0, ROWS_PER_TILE)])

  return layer


_layer = _make_layer()


def kernel(edge_index, edge_weight, user_emb, item_emb):
  all_emb = jnp.concatenate([user_emb, item_emb], axis=0)
  pad = E_PAD - E
  src = jnp.concatenate([edge_index[1], jnp.zeros((pad,), jnp.int32)])
  dst = jnp.concatenate([edge_index[0], jnp.zeros((pad,), jnp.int32)])
  w = jnp.concatenate([edge_weight, jnp.zeros((pad,), jnp.float32)])
  zeros = jnp.zeros((N, D), jnp.float32)

  acc = all_emb
  emb = all_emb
  for _ in range(N_LAYERS):
    parts = _layer(src, dst, w, emb, zeros)
    emb = parts[0] + parts[1]
    acc = acc + emb
  out = acc * (1.0 / (N_LAYERS + 1))
  return out[:NUM_USERS], out[NUM_USERS:]


# speedup vs baseline: 5.8471x; 5.8471x over previous
import functools

import jax
import jax.numpy as jnp
from jax import lax
from jax.experimental import pallas as pl
from jax.experimental.pallas import tpu as pltpu
from jax.experimental.pallas import tpu_sc as plsc

NUM_USERS = 25000
NUM_ITEMS = 25000
N = NUM_USERS + NUM_ITEMS
E = 1600000
D = 16
N_LAYERS = 3

NUM_CORES = 2
NUM_SUBCORES = 16
NUM_WORKERS = NUM_CORES * NUM_SUBCORES
CHUNK = 128
CHUNKS_PER_TILE = -(-E // (NUM_WORKERS * CHUNK))
E_PAD = NUM_WORKERS * CHUNK * CHUNKS_PER_TILE
ROWS_PER_TILE = N // NUM_SUBCORES
UNROLL = 8


def _make_layer():
  mesh = plsc.VectorSubcoreMesh(core_axis_name="c", subcore_axis_name="s")

  @functools.partial(
      pl.kernel,
      mesh=mesh,
      compiler_params=pltpu.CompilerParams(
          use_tc_tiling_on_sc=False, needs_layout_passes=False),
      out_type=jax.ShapeDtypeStruct((NUM_CORES, N, D), jnp.float32),
      scratch_types=[
          pltpu.VMEM((CHUNK,), jnp.int32),
          pltpu.VMEM((CHUNK,), jnp.int32),
          pltpu.VMEM((CHUNK,), jnp.float32),
          pltpu.VMEM((CHUNK, D), jnp.float32),
          pltpu.VMEM_SHARED((N, D), jnp.float32),
          pltpu.SemaphoreType.DMA,
      ],
  )
  def layer(src_hbm, dst_hbm, w_hbm, emb_hbm, zeros_hbm, out_hbm,
            src_v, dst_v, w_v, rows_v, acc, sem):
    cid = lax.axis_index("c")
    sid = lax.axis_index("s")
    wid = sid * NUM_CORES + cid

    r0 = sid * ROWS_PER_TILE
    pltpu.sync_copy(zeros_hbm.at[pl.ds(r0, ROWS_PER_TILE)],
                    acc.at[pl.ds(r0, ROWS_PER_TILE)])
    plsc.subcore_barrier()

    def chunk_body(c, carry):
      base = (wid * CHUNKS_PER_TILE + c) * CHUNK
      pltpu.sync_copy(src_hbm.at[pl.ds(base, CHUNK)], src_v)
      pltpu.sync_copy(dst_hbm.at[pl.ds(base, CHUNK)], dst_v)
      pltpu.sync_copy(w_hbm.at[pl.ds(base, CHUNK)], w_v)
      pltpu.async_copy(emb_hbm.at[src_v], rows_v, sem).wait()

      def mul_body(g, carry2):
        for j in range(UNROLL):
          k = g * UNROLL + j
          wb = plsc.load_gather(w_v, [jnp.full((16,), k, jnp.int32)])
          rows_v[k, :] = rows_v[k, :] * wb
        return carry2

      lax.fori_loop(0, CHUNK // UNROLL, mul_body, 0, unroll=False)
      pltpu.sync_copy(rows_v, acc.at[dst_v], add=True)
      return carry

    lax.fori_loop(0, CHUNKS_PER_TILE, chunk_body, 0, unroll=False)
    plsc.subcore_barrier()

    pltpu.sync_copy(acc.at[pl.ds(r0, ROWS_PER_TILE)],
                    out_hbm.at[cid, pl.ds(r0, ROWS_PER_TILE)])

  return layer


_layer = _make_layer()


def kernel(edge_index, edge_weight, user_emb, item_emb):
  all_emb = jnp.concatenate([user_emb, item_emb], axis=0)
  pad = E_PAD - E
  src = jnp.concatenate([edge_index[1], jnp.zeros((pad,), jnp.int32)])
  dst = jnp.concatenate([edge_index[0], jnp.zeros((pad,), jnp.int32)])
  w = jnp.concatenate([edge_weight, jnp.zeros((pad,), jnp.float32)])
  zeros = jnp.zeros((N, D), jnp.float32)

  acc = all_emb
  emb = all_emb
  for _ in range(N_LAYERS):
    parts = _layer(src, dst, w, emb, zeros)
    emb = parts[0] + parts[1]
    acc = acc + emb
  out = acc * (1.0 / (N_LAYERS + 1))
  return out[:NUM_USERS], out[NUM_USERS:]


# speedup vs baseline: 13.7823x; 2.3571x over previous
import functools

import jax
import jax.numpy as jnp
from jax import lax
from jax.experimental import pallas as pl
from jax.experimental.pallas import tpu as pltpu
from jax.experimental.pallas import tpu_sc as plsc

NUM_USERS = 25000
NUM_ITEMS = 25000
N = NUM_USERS + NUM_ITEMS
E = 1600000
D = 16
N_LAYERS = 3

NUM_CORES = 2
NUM_SUBCORES = 16
NUM_WORKERS = NUM_CORES * NUM_SUBCORES
CHUNK = 128
RING = 8
SUBBLOCKS = 7
SUPERBLOCKS = 7
CHUNKS_PER_TILE = RING * SUBBLOCKS * SUPERBLOCKS
SB_EDGES = RING * SUBBLOCKS * CHUNK
TILE_EDGES = CHUNKS_PER_TILE * CHUNK
E_PAD = NUM_WORKERS * TILE_EDGES
ROWS_PER_TILE = N // NUM_SUBCORES
MUL_UNROLL = 16


def _make_layer():
  mesh = plsc.VectorSubcoreMesh(core_axis_name="c", subcore_axis_name="s")

  @functools.partial(
      pl.kernel,
      mesh=mesh,
      compiler_params=pltpu.CompilerParams(
          use_tc_tiling_on_sc=False, needs_layout_passes=False),
      out_type=jax.ShapeDtypeStruct((NUM_CORES, N, D), jnp.float32),
      scratch_types=[
          pltpu.VMEM((SB_EDGES,), jnp.int32),
          pltpu.VMEM((SB_EDGES,), jnp.float32),
          [pltpu.VMEM((CHUNK,), jnp.int32) for _ in range(RING)],
          pltpu.VMEM((RING, CHUNK, D), jnp.float32),
          pltpu.VMEM_SHARED((N, D), jnp.float32),
          pltpu.SemaphoreType.DMA,
          pltpu.SemaphoreType.DMA,
          pltpu.SemaphoreType.DMA,
      ],
  )
  def layer(src_hbm, dst_hbm, w_hbm, emb_hbm, zeros_hbm, out_hbm,
            src_v, w_v, dst_ring, rows, acc, sem_g, sem_s, sem_i):
    cid = lax.axis_index("c")
    sid = lax.axis_index("s")
    wid = sid * NUM_CORES + cid

    r0 = sid * ROWS_PER_TILE
    pltpu.sync_copy(zeros_hbm.at[pl.ds(r0, ROWS_PER_TILE)],
                    acc.at[pl.ds(r0, ROWS_PER_TILE)])
    plsc.subcore_barrier()

    tile_base = wid * TILE_EDGES

    def superblock_body(sb, carry):
      sb_base = tile_base + sb * SB_EDGES
      pltpu.sync_copy(src_hbm.at[pl.ds(sb_base, SB_EDGES)], src_v)
      pltpu.sync_copy(w_hbm.at[pl.ds(sb_base, SB_EDGES)], w_v)

      def subblock_body(s, carry2):
        for j in range(RING):
          off = (s * RING + j) * CHUNK
          pltpu.async_copy(dst_hbm.at[pl.ds(sb_base + off, CHUNK)],
                           dst_ring[j], sem_i)
          pltpu.async_copy(emb_hbm.at[src_v.at[pl.ds(off, CHUNK)]],
                           rows.at[j], sem_g)
        for j in range(RING):
          off = (s * RING + j) * CHUNK
          pltpu.make_async_copy(emb_hbm.at[src_v.at[pl.ds(off, CHUNK)]],
                                rows.at[j], sem_g).wait()

          def mul_body(g, carry3, j=j, off=off):
            for u in range(MUL_UNROLL):
              k = g * MUL_UNROLL + u
              wb = plsc.load_gather(w_v, [jnp.full((16,), off + k, jnp.int32)])
              rows[j, k, :] = rows[j, k, :] * wb
            return carry3

          lax.fori_loop(0, CHUNK // MUL_UNROLL, mul_body, 0, unroll=False)
          pltpu.make_async_copy(dst_hbm.at[pl.ds(sb_base + off, CHUNK)],
                                dst_ring[j], sem_i).wait()
          pltpu.async_copy(rows.at[j], acc.at[dst_ring[j]], sem_s, add=True)
        for j in range(RING):
          pltpu.make_async_copy(rows.at[j], acc.at[dst_ring[j]], sem_s).wait()
        return carry2

      lax.fori_loop(0, SUBBLOCKS, subblock_body, 0, unroll=False)
      return carry

    lax.fori_loop(0, SUPERBLOCKS, superblock_body, 0, unroll=False)
    plsc.subcore_barrier()

    pltpu.sync_copy(acc.at[pl.ds(r0, ROWS_PER_TILE)],
                    out_hbm.at[cid, pl.ds(r0, ROWS_PER_TILE)])

  return layer


_layer = _make_layer()


def kernel(edge_index, edge_weight, user_emb, item_emb):
  all_emb = jnp.concatenate([user_emb, item_emb], axis=0)
  pad = E_PAD - E
  src = jnp.concatenate([edge_index[1], jnp.zeros((pad,), jnp.int32)])
  dst = jnp.concatenate([edge_index[0], jnp.zeros((pad,), jnp.int32)])
  w = jnp.concatenate([edge_weight, jnp.zeros((pad,), jnp.float32)])
  zeros = jnp.zeros((N, D), jnp.float32)

  acc = all_emb
  emb = all_emb
  for _ in range(N_LAYERS):
    parts = _layer(src, dst, w, emb, zeros)
    emb = parts[0] + parts[1]
    acc = acc + emb
  out = acc * (1.0 / (N_LAYERS + 1))
  return out[:NUM_USERS], out[NUM_USERS:]


# speedup vs baseline: 25.2430x; 1.8315x over previous
import functools

import jax
import jax.numpy as jnp
from jax import lax
from jax.experimental import pallas as pl
from jax.experimental.pallas import tpu as pltpu
from jax.experimental.pallas import tpu_sc as plsc

NUM_USERS = 25000
NUM_ITEMS = 25000
N = NUM_USERS + NUM_ITEMS
E = 1600000
D = 16
N_LAYERS = 3

NUM_CORES = 2
NUM_SUBCORES = 16
NUM_WORKERS = NUM_CORES * NUM_SUBCORES
CHUNK = 128
RING = 8
SUBBLOCKS = 7
SUPERBLOCKS = 7
CHUNKS_PER_TILE = RING * SUBBLOCKS * SUPERBLOCKS
SB_EDGES = RING * SUBBLOCKS * CHUNK
TILE_EDGES = CHUNKS_PER_TILE * CHUNK
E_PAD = NUM_WORKERS * TILE_EDGES
ROWS_PER_TILE = N // NUM_SUBCORES
MUL_UNROLL = 16


def _make_layer():
  mesh = plsc.VectorSubcoreMesh(core_axis_name="c", subcore_axis_name="s")

  @functools.partial(
      pl.kernel,
      mesh=mesh,
      compiler_params=pltpu.CompilerParams(
          use_tc_tiling_on_sc=False, needs_layout_passes=False),
      out_type=jax.ShapeDtypeStruct((NUM_CORES, N, D), jnp.float32),
      scratch_types=[
          pltpu.VMEM((SB_EDGES,), jnp.int32),
          pltpu.VMEM((SB_EDGES,), jnp.float32),
          [pltpu.VMEM((CHUNK,), jnp.int32) for _ in range(RING)],
          pltpu.VMEM((RING, CHUNK, D), jnp.float32),
          pltpu.VMEM_SHARED((N, D), jnp.float32),
          pltpu.SemaphoreType.DMA,
          pltpu.SemaphoreType.DMA,
          pltpu.SemaphoreType.DMA,
      ],
  )
  def layer(src_hbm, dst_hbm, w_hbm, emb_hbm, zeros_hbm, out_hbm,
            src_v, w_v, dst_ring, rows, acc, sem_g, sem_s, sem_i):
    cid = lax.axis_index("c")
    sid = lax.axis_index("s")
    wid = sid * NUM_CORES + cid

    r0 = sid * ROWS_PER_TILE
    pltpu.sync_copy(zeros_hbm.at[pl.ds(r0, ROWS_PER_TILE)],
                    acc.at[pl.ds(r0, ROWS_PER_TILE)])
    plsc.subcore_barrier()

    tile_base = wid * TILE_EDGES

    def superblock_body(sb, carry):
      sb_base = tile_base + sb * SB_EDGES
      pltpu.sync_copy(src_hbm.at[pl.ds(sb_base, SB_EDGES)], src_v)
      pltpu.sync_copy(w_hbm.at[pl.ds(sb_base, SB_EDGES)], w_v)

      def subblock_body(s, carry2):
        for j in range(RING):
          off = (s * RING + j) * CHUNK
          pltpu.async_copy(dst_hbm.at[pl.ds(sb_base + off, CHUNK)],
                           dst_ring[j], sem_i)
          pltpu.async_copy(emb_hbm.at[src_v.at[pl.ds(off, CHUNK)]],
                           rows.at[j], sem_g)
        for j in range(RING):
          off = (s * RING + j) * CHUNK
          pltpu.make_async_copy(emb_hbm.at[src_v.at[pl.ds(off, CHUNK)]],
                                rows.at[j], sem_g).wait()

          def mul_body(g, carry3, j=j, off=off):
            w16 = w_v[pl.ds(off + g * MUL_UNROLL, MUL_UNROLL)]
            for u in range(MUL_UNROLL):
              k = g * MUL_UNROLL + u
              wb = w16.at[jnp.full((16,), u, jnp.int32)].get(
                  mode="promise_in_bounds")
              rows[j, k, :] = rows[j, k, :] * wb
            return carry3

          lax.fori_loop(0, CHUNK // MUL_UNROLL, mul_body, 0, unroll=False)
          pltpu.make_async_copy(dst_hbm.at[pl.ds(sb_base + off, CHUNK)],
                                dst_ring[j], sem_i).wait()
          pltpu.async_copy(rows.at[j], acc.at[dst_ring[j]], sem_s, add=True)
        for j in range(RING):
          pltpu.make_async_copy(rows.at[j], acc.at[dst_ring[j]], sem_s).wait()
        return carry2

      lax.fori_loop(0, SUBBLOCKS, subblock_body, 0, unroll=False)
      return carry

    lax.fori_loop(0, SUPERBLOCKS, superblock_body, 0, unroll=False)
    plsc.subcore_barrier()

    pltpu.sync_copy(acc.at[pl.ds(r0, ROWS_PER_TILE)],
                    out_hbm.at[cid, pl.ds(r0, ROWS_PER_TILE)])

  return layer


_layer = _make_layer()


def kernel(edge_index, edge_weight, user_emb, item_emb):
  all_emb = jnp.concatenate([user_emb, item_emb], axis=0)
  pad = E_PAD - E
  src = jnp.concatenate([edge_index[1], jnp.zeros((pad,), jnp.int32)])
  dst = jnp.concatenate([edge_index[0], jnp.zeros((pad,), jnp.int32)])
  w = jnp.concatenate([edge_weight, jnp.zeros((pad,), jnp.float32)])
  zeros = jnp.zeros((N, D), jnp.float32)

  acc = all_emb
  emb = all_emb
  for _ in range(N_LAYERS):
    parts = _layer(src, dst, w, emb, zeros)
    emb = parts[0] + parts[1]
    acc = acc + emb
  out = acc * (1.0 / (N_LAYERS + 1))
  return out[:NUM_USERS], out[NUM_USERS:]


# speedup vs baseline: 28.6108x; 1.1334x over previous
import functools

import jax
import jax.numpy as jnp
from jax import lax
from jax.experimental import pallas as pl
from jax.experimental.pallas import tpu as pltpu
from jax.experimental.pallas import tpu_sc as plsc

NUM_USERS = 25000
NUM_ITEMS = 25000
N = NUM_USERS + NUM_ITEMS
E = 1600000
D = 16
N_LAYERS = 3

NUM_CORES = 2
NUM_SUBCORES = 16
NUM_WORKERS = NUM_CORES * NUM_SUBCORES
TILE_EDGES = E // NUM_WORKERS
CHUNK = 128
RING = 8
SB_CHUNKS = 64
SB_EDGES = SB_CHUNKS * CHUNK
N_SB = 6
TAIL_CHUNKS = [128] * 6 + [80]
TAIL_BASE = N_SB * SB_EDGES
ROWS_PER_TILE = N // NUM_SUBCORES
MUL_UNROLL = 16


def _scale_rows(rows, w_v, j, off, n_edges):
  def mul_body(g, carry):
    w16 = w_v[pl.ds(off + g * MUL_UNROLL, MUL_UNROLL)]
    for u in range(MUL_UNROLL):
      k = g * MUL_UNROLL + u
      wb = w16.at[jnp.full((16,), u, jnp.int32)].get(mode="promise_in_bounds")
      rows[j, k, :] = rows[j, k, :] * wb
    return carry

  lax.fori_loop(0, n_edges // MUL_UNROLL, mul_body, 0, unroll=False)


def _make_layer():
  mesh = plsc.VectorSubcoreMesh(core_axis_name="c", subcore_axis_name="s")

  @functools.partial(
      pl.kernel,
      mesh=mesh,
      compiler_params=pltpu.CompilerParams(
          use_tc_tiling_on_sc=False, needs_layout_passes=False),
      out_type=jax.ShapeDtypeStruct((NUM_CORES, N, D), jnp.float32),
      scratch_types=[
          pltpu.VMEM((SB_EDGES,), jnp.int32),
          pltpu.VMEM((SB_EDGES,), jnp.float32),
          [pltpu.VMEM((CHUNK,), jnp.int32) for _ in range(RING)],
          pltpu.VMEM((80,), jnp.int32),
          pltpu.VMEM((RING, CHUNK, D), jnp.float32),
          pltpu.VMEM_SHARED((N, D), jnp.float32),
          pltpu.SemaphoreType.DMA,
          pltpu.SemaphoreType.DMA,
          pltpu.SemaphoreType.DMA,
      ],
  )
  def layer(edge_hbm, w_hbm, emb_hbm, zeros_hbm, out_hbm,
            src_v, w_v, dst_ring, dst_tail, rows, acc, sem_g, sem_s, sem_i):
    cid = lax.axis_index("c")
    sid = lax.axis_index("s")
    wid = sid * NUM_CORES + cid

    r0 = sid * ROWS_PER_TILE
    pltpu.sync_copy(zeros_hbm.at[pl.ds(r0, ROWS_PER_TILE)],
                    acc.at[pl.ds(r0, ROWS_PER_TILE)])
    plsc.subcore_barrier()

    tile_base = wid * TILE_EDGES

    def run_wave(sb_base, offs_sizes):
      for j, (off, size) in enumerate(offs_sizes):
        dref = dst_tail if size != CHUNK else dst_ring[j]
        pltpu.async_copy(edge_hbm.at[0, pl.ds(sb_base + off, size)],
                         dref, sem_i)
        pltpu.async_copy(emb_hbm.at[src_v.at[pl.ds(off, size)]],
                         rows.at[j, pl.ds(0, size)], sem_g)
      for j, (off, size) in enumerate(offs_sizes):
        dref = dst_tail if size != CHUNK else dst_ring[j]
        pltpu.make_async_copy(emb_hbm.at[src_v.at[pl.ds(off, size)]],
                              rows.at[j, pl.ds(0, size)], sem_g).wait()
        _scale_rows(rows, w_v, j, off, size)
        pltpu.make_async_copy(edge_hbm.at[0, pl.ds(sb_base + off, size)],
                              dref, sem_i).wait()
        pltpu.async_copy(rows.at[j, pl.ds(0, size)], acc.at[dref], sem_s,
                         add=True)
      for j, (off, size) in enumerate(offs_sizes):
        dref = dst_tail if size != CHUNK else dst_ring[j]
        pltpu.make_async_copy(rows.at[j, pl.ds(0, size)], acc.at[dref],
                              sem_s).wait()

    def superblock_body(sb, carry):
      sb_base = tile_base + sb * SB_EDGES
      pltpu.sync_copy(edge_hbm.at[1, pl.ds(sb_base, SB_EDGES)], src_v)
      pltpu.sync_copy(w_hbm.at[pl.ds(sb_base, SB_EDGES)], w_v)

      def subblock_body(s, carry2):
        run_wave(sb_base, [(s * RING * CHUNK + j * CHUNK, CHUNK)
                           for j in range(RING)])
        return carry2

      lax.fori_loop(0, SB_CHUNKS // RING, subblock_body, 0, unroll=False)
      return carry

    lax.fori_loop(0, N_SB, superblock_body, 0, unroll=False)

    tail_base = tile_base + TAIL_BASE
    n_tail = sum(TAIL_CHUNKS)
    pltpu.sync_copy(edge_hbm.at[1, pl.ds(tail_base, n_tail)],
                    src_v.at[pl.ds(0, n_tail)])
    pltpu.sync_copy(w_hbm.at[pl.ds(tail_base, n_tail)],
                    w_v.at[pl.ds(0, n_tail)])
    offs = []
    o = 0
    for size in TAIL_CHUNKS:
      offs.append((o, size))
      o += size
    run_wave(tail_base, offs)

    plsc.subcore_barrier()

    pltpu.sync_copy(acc.at[pl.ds(r0, ROWS_PER_TILE)],
                    out_hbm.at[cid, pl.ds(r0, ROWS_PER_TILE)])

  return layer


_layer = _make_layer()


def kernel(edge_index, edge_weight, user_emb, item_emb):
  all_emb = jnp.concatenate([user_emb, item_emb], axis=0)
  zeros = jnp.zeros((N, D), jnp.float32)

  acc = all_emb
  emb = all_emb
  for _ in range(N_LAYERS):
    parts = _layer(edge_index, edge_weight, emb, zeros)
    emb = parts[0] + parts[1]
    acc = acc + emb
  out = acc * (1.0 / (N_LAYERS + 1))
  return out[:NUM_USERS], out[NUM_USERS:]


# speedup vs baseline: 31.1650x; 1.0893x over previous
import functools

import jax
import jax.numpy as jnp
from jax import lax
from jax.experimental import pallas as pl
from jax.experimental.pallas import tpu as pltpu
from jax.experimental.pallas import tpu_sc as plsc

NUM_USERS = 25000
NUM_ITEMS = 25000
N = NUM_USERS + NUM_ITEMS
E = 1600000
D = 16
N_LAYERS = 3

NUM_CORES = 2
NUM_SUBCORES = 16
NUM_WORKERS = NUM_CORES * NUM_SUBCORES
TILE_EDGES = E // NUM_WORKERS
CHUNK = 128
RING = 8
SB_CHUNKS = 64
SB_EDGES = SB_CHUNKS * CHUNK
N_SB = 6
TAIL_CHUNKS = [128] * 6 + [80]
TAIL_BASE = N_SB * SB_EDGES
ROWS_PER_TILE = N // NUM_SUBCORES
MUL_UNROLL = 16
PASS_ROWS = [(0, 1440), (1440, 1440), (2880, 245)]
BUF_ROWS = 1440


def _scale_rows(rows, w_v, j, off, n_edges):
  def mul_body(g, carry):
    w16 = w_v[pl.ds(off + g * MUL_UNROLL, MUL_UNROLL)]
    for u in range(MUL_UNROLL):
      k = g * MUL_UNROLL + u
      wb = w16.at[jnp.full((16,), u, jnp.int32)].get(mode="promise_in_bounds")
      rows[j, k, :] = rows[j, k, :] * wb
    return carry

  lax.fori_loop(0, n_edges // MUL_UNROLL, mul_body, 0, unroll=False)


def _add_rows(buf_a, buf_b, n_rows, unroll):
  def body(g, carry):
    for u in range(unroll):
      i = g * unroll + u
      buf_a[i, :] = buf_a[i, :] + buf_b[i, :]
    return carry

  lax.fori_loop(0, n_rows // unroll, body, 0, unroll=False)


def _zero_buf(buf, n_rows, unroll):
  z = jnp.zeros((16,), jnp.float32)

  def body(g, carry):
    for u in range(unroll):
      buf[g * unroll + u, :] = z
    return carry

  lax.fori_loop(0, n_rows // unroll, body, 0, unroll=False)


def _edge_phase(edge_src_hbm, edge_dst_hbm, w_hbm, comb, acc,
                src_v, w_v, dst_ring, dst_tail, rows, sem_g, sem_s, sem_i,
                tile_base):
  def run_wave(sb_base, offs_sizes):
    for j, (off, size) in enumerate(offs_sizes):
      dref = dst_tail if size != CHUNK else dst_ring[j]
      pltpu.async_copy(edge_dst_hbm.at[pl.ds(sb_base + off, size)],
                       dref, sem_i)
      pltpu.async_copy(comb.at[src_v.at[pl.ds(off, size)]],
                       rows.at[j, pl.ds(0, size)], sem_g)
    for j, (off, size) in enumerate(offs_sizes):
      dref = dst_tail if size != CHUNK else dst_ring[j]
      pltpu.make_async_copy(comb.at[src_v.at[pl.ds(off, size)]],
                            rows.at[j, pl.ds(0, size)], sem_g).wait()
      _scale_rows(rows, w_v, j, off, size)
      pltpu.make_async_copy(edge_dst_hbm.at[pl.ds(sb_base + off, size)],
                            dref, sem_i).wait()
      pltpu.async_copy(rows.at[j, pl.ds(0, size)], acc.at[dref], sem_s,
                       add=True)
    for j, (off, size) in enumerate(offs_sizes):
      dref = dst_tail if size != CHUNK else dst_ring[j]
      pltpu.make_async_copy(rows.at[j, pl.ds(0, size)], acc.at[dref],
                            sem_s).wait()

  def superblock_body(sb, carry):
    sb_base = tile_base + sb * SB_EDGES
    pltpu.sync_copy(edge_src_hbm.at[pl.ds(sb_base, SB_EDGES)], src_v)
    pltpu.sync_copy(w_hbm.at[pl.ds(sb_base, SB_EDGES)], w_v)

    def subblock_body(s, carry2):
      run_wave(sb_base, [(s * RING * CHUNK + j * CHUNK, CHUNK)
                         for j in range(RING)])
      return carry2

    lax.fori_loop(0, SB_CHUNKS // RING, subblock_body, 0, unroll=False)
    return carry

  lax.fori_loop(0, N_SB, superblock_body, 0, unroll=False)

  tail_base = tile_base + TAIL_BASE
  n_tail = sum(TAIL_CHUNKS)
  pltpu.sync_copy(edge_src_hbm.at[pl.ds(tail_base, n_tail)],
                  src_v.at[pl.ds(0, n_tail)])
  pltpu.sync_copy(w_hbm.at[pl.ds(tail_base, n_tail)],
                  w_v.at[pl.ds(0, n_tail)])
  offs = []
  o = 0
  for size in TAIL_CHUNKS:
    offs.append((o, size))
    o += size
  run_wave(tail_base, offs)


_SCRATCH = [
    pltpu.VMEM((SB_EDGES,), jnp.int32),
    pltpu.VMEM((SB_EDGES,), jnp.float32),
    [pltpu.VMEM((CHUNK,), jnp.int32) for _ in range(RING)],
    pltpu.VMEM((80,), jnp.int32),
    pltpu.VMEM((RING, CHUNK, D), jnp.float32),
    pltpu.VMEM((BUF_ROWS, D), jnp.float32),
    pltpu.VMEM((BUF_ROWS, D), jnp.float32),
    pltpu.VMEM_SHARED((N, D), jnp.float32),
    pltpu.SemaphoreType.DMA,
    pltpu.SemaphoreType.DMA,
    pltpu.SemaphoreType.DMA,
]

_PARAMS = dict(
    mesh=plsc.VectorSubcoreMesh(core_axis_name="c", subcore_axis_name="s"),
    compiler_params=pltpu.CompilerParams(
        use_tc_tiling_on_sc=False, needs_layout_passes=False),
    scratch_types=_SCRATCH,
)


@functools.partial(
    pl.kernel,
    out_type=(jax.ShapeDtypeStruct((NUM_CORES, N, D), jnp.float32),
              jax.ShapeDtypeStruct((NUM_CORES, N, D), jnp.float32)),
    **_PARAMS,
)
def _layer_first(src_hbm, dst_hbm, w_hbm, user_hbm, item_hbm,
                 out_hbm, comb_hbm,
                 src_v, w_v, dst_ring, dst_tail, rows, buf_a, buf_b,
                 acc, sem_g, sem_s, sem_i):
  cid = lax.axis_index("c")
  sid = lax.axis_index("s")
  wid = sid * NUM_CORES + cid
  r0 = sid * ROWS_PER_TILE

  half = NUM_SUBCORES // 2
  for poff, prows in PASS_ROWS:

    @pl.when(sid < half)
    def _(poff=poff, prows=prows):
      pltpu.sync_copy(user_hbm.at[pl.ds(r0 + poff, prows)],
                      buf_a.at[pl.ds(0, prows)])

    @pl.when(sid >= half)
    def _(poff=poff, prows=prows):
      pltpu.sync_copy(
          item_hbm.at[pl.ds((sid - half) * ROWS_PER_TILE + poff, prows)],
          buf_a.at[pl.ds(0, prows)])

    pltpu.sync_copy(buf_a.at[pl.ds(0, prows)],
                    comb_hbm.at[cid, pl.ds(r0 + poff, prows)])

  _zero_buf(buf_b, BUF_ROWS, 8)
  for poff, prows in PASS_ROWS:
    pltpu.sync_copy(buf_b.at[pl.ds(0, prows)], acc.at[pl.ds(r0 + poff, prows)])
  plsc.subcore_barrier()

  _edge_phase(src_hbm, dst_hbm, w_hbm, comb_hbm.at[cid], acc,
              src_v, w_v, dst_ring, dst_tail, rows, sem_g, sem_s, sem_i,
              wid * TILE_EDGES)
  plsc.subcore_barrier()

  pltpu.sync_copy(acc.at[pl.ds(r0, ROWS_PER_TILE)],
                  out_hbm.at[cid, pl.ds(r0, ROWS_PER_TILE)])


@functools.partial(
    pl.kernel,
    out_type=(jax.ShapeDtypeStruct((NUM_CORES, N, D), jnp.float32),
              jax.ShapeDtypeStruct((NUM_CORES, N, D), jnp.float32)),
    **_PARAMS,
)
def _layer_next(src_hbm, dst_hbm, w_hbm, pprev_hbm, out_hbm, comb_hbm,
                src_v, w_v, dst_ring, dst_tail, rows, buf_a, buf_b,
                acc, sem_g, sem_s, sem_i):
  cid = lax.axis_index("c")
  sid = lax.axis_index("s")
  wid = sid * NUM_CORES + cid
  r0 = sid * ROWS_PER_TILE

  for poff, prows in PASS_ROWS:
    pltpu.sync_copy(pprev_hbm.at[0, pl.ds(r0 + poff, prows)],
                    buf_a.at[pl.ds(0, prows)])
    pltpu.sync_copy(pprev_hbm.at[1, pl.ds(r0 + poff, prows)],
                    buf_b.at[pl.ds(0, prows)])
    _add_rows(buf_a, buf_b, prows, 8 if prows % 8 == 0 else 5)
    pltpu.sync_copy(buf_a.at[pl.ds(0, prows)],
                    comb_hbm.at[cid, pl.ds(r0 + poff, prows)])

  _zero_buf(buf_b, BUF_ROWS, 8)
  for poff, prows in PASS_ROWS:
    pltpu.sync_copy(buf_b.at[pl.ds(0, prows)], acc.at[pl.ds(r0 + poff, prows)])
  plsc.subcore_barrier()

  _edge_phase(src_hbm, dst_hbm, w_hbm, comb_hbm.at[cid], acc,
              src_v, w_v, dst_ring, dst_tail, rows, sem_g, sem_s, sem_i,
              wid * TILE_EDGES)
  plsc.subcore_barrier()

  pltpu.sync_copy(acc.at[pl.ds(r0, ROWS_PER_TILE)],
                  out_hbm.at[cid, pl.ds(r0, ROWS_PER_TILE)])


def kernel(edge_index, edge_weight, user_emb, item_emb):
  src = edge_index[1]
  dst = edge_index[0]

  p0, _ = _layer_first(src, dst, edge_weight, user_emb, item_emb)
  p1, c1 = _layer_next(src, dst, edge_weight, p0)
  p2, c2 = _layer_next(src, dst, edge_weight, p1)

  quarter = jnp.float32(0.25)
  users = (user_emb + c1[0, :NUM_USERS] + c2[0, :NUM_USERS]
           + p2[0, :NUM_USERS] + p2[1, :NUM_USERS]) * quarter
  items = (item_emb + c1[0, NUM_USERS:] + c2[0, NUM_USERS:]
           + p2[0, NUM_USERS:] + p2[1, NUM_USERS:]) * quarter
  return users, items
